# baseline (speedup 1.0000x reference)
import functools

import jax
import jax.numpy as jnp
from jax import lax
from jax.experimental import pallas as pl
from jax.experimental.pallas import tpu as pltpu
from jax.experimental.pallas import tpu_sc as plsc

NC = 2
NS = 16
NW = NC * NS
CHUNK = 128
LANES = 16

_mesh = plsc.VectorSubcoreMesh(
    core_axis_name="c", subcore_axis_name="s", num_cores=NC, num_subcores=NS
)
_sc_params = pltpu.CompilerParams(needs_layout_passes=False)


def _make_deg_kernel(n_rows, n_chunks):
    @functools.partial(
        pl.kernel,
        out_type=jax.ShapeDtypeStruct((NW, n_rows, LANES), jnp.float32),
        mesh=_mesh,
        scratch_types=[
            pltpu.VMEM((n_chunks, CHUNK), jnp.int32),
            pltpu.VMEM((n_chunks, CHUNK), jnp.float32),
            pltpu.VMEM((n_rows, LANES), jnp.float32),
        ],
        compiler_params=_sc_params,
    )
    def deg_kernel(dst_hbm, w_hbm, out_hbm, dst_v, w_v, pdeg):
        c = lax.axis_index("c")
        s = lax.axis_index("s")
        wid = s * NC + c
        pltpu.sync_copy(dst_hbm.at[wid], dst_v)
        pltpu.sync_copy(w_hbm.at[wid], w_v)

        zero16 = jnp.zeros((LANES,), jnp.float32)

        @pl.loop(0, n_rows)
        def _zero_pdeg(r):
            pdeg.at[r, pl.ds(0, LANES)][...] = zero16

        @pl.loop(0, n_chunks)
        def _edges(ch):
            for g in range(CHUNK // LANES):
                dst16 = dst_v[ch, pl.ds(g * LANES, LANES)]
                w16 = w_v[ch, pl.ds(g * LANES, LANES)]
                plsc.addupdate_scatter(
                    pdeg, [dst16 >> 4, dst16 & (LANES - 1)], w16)

        pltpu.sync_copy(pdeg, out_hbm.at[wid])

    return deg_kernel


def _make_msg_kernel(n_pad, d, n_chunks):
    rows_per_tile = n_pad // NS

    @functools.partial(
        pl.kernel,
        out_type=jax.ShapeDtypeStruct((NC, n_pad, d), jnp.float32),
        mesh=_mesh,
        scratch_types=[
            pltpu.VMEM((2, CHUNK), jnp.int32),
            pltpu.VMEM((2, CHUNK), jnp.int32),
            pltpu.VMEM((2, CHUNK), jnp.float32),
            pltpu.VMEM((CHUNK, d), jnp.float32),
            pltpu.VMEM((CHUNK, d), jnp.float32),
            pltpu.VMEM_SHARED((n_pad, d), jnp.float32),
            pltpu.SemaphoreType.DMA,
            pltpu.SemaphoreType.DMA,
            pltpu.SemaphoreType.DMA,
            pltpu.SemaphoreType.DMA,
        ],
        compiler_params=_sc_params,
    )
    def msg_kernel(hp_hbm, src_hbm, dst_hbm, w_hbm, out_hbm,
                   sidx_v, didx_v, wv2, rows0_v, rows1_v, acc_sh,
                   sem_g0, sem_g1, sem_i0, sem_i1):
        c = lax.axis_index("c")
        s = lax.axis_index("s")
        wid = s * NC + c

        zero16 = jnp.zeros((LANES,), jnp.float32)

        @pl.loop(0, CHUNK)
        def _zero_rows(r):
            for q in range(d // LANES):
                rows0_v.at[r, pl.ds(q * LANES, LANES)][...] = zero16

        base = s * rows_per_tile
        off = 0
        while off < rows_per_tile:
            sz = min(CHUNK, rows_per_tile - off)
            pltpu.sync_copy(
                rows0_v.at[pl.ds(0, sz)], acc_sh.at[pl.ds(base + off, sz)]
            )
            off += sz
        plsc.subcore_barrier()

        rows = (rows0_v, rows1_v)
        sem_g = (sem_g0, sem_g1)
        sem_i = (sem_i0, sem_i1)

        def prefetch_idx(b, ch):
            pltpu.async_copy(src_hbm.at[wid, ch], sidx_v.at[b], sem_i[b])
            pltpu.async_copy(dst_hbm.at[wid, ch], didx_v.at[b], sem_i[b])
            pltpu.async_copy(w_hbm.at[wid, ch], wv2.at[b], sem_i[b])

        def wait_idx(b):
            pltpu.make_async_copy(src_hbm.at[wid, 0], sidx_v.at[b],
                                  sem_i[b]).wait()
            pltpu.make_async_copy(dst_hbm.at[wid, 0], didx_v.at[b],
                                  sem_i[b]).wait()
            pltpu.make_async_copy(w_hbm.at[wid, 0], wv2.at[b],
                                  sem_i[b]).wait()

        def scale(b):
            rw = rows[b]

            @pl.loop(0, CHUNK)
            def _scale(j):
                jidx = jnp.full((LANES,), j, dtype=jnp.int32)
                w16 = plsc.load_gather(wv2.at[b], [jidx])
                for q in range(d // LANES):
                    sl = rw.at[j, pl.ds(q * LANES, LANES)]
                    sl[...] = sl[...] * w16

        prefetch_idx(0, 0)
        prefetch_idx(1, 1)
        wait_idx(0)
        pltpu.async_copy(hp_hbm.at[sidx_v.at[0]], rows0_v, sem_g0)

        def body(b, ch):
            wait_idx(1 - b)
            pltpu.async_copy(hp_hbm.at[sidx_v.at[1 - b]], rows[1 - b],
                             sem_g[1 - b])
            pltpu.make_async_copy(hp_hbm.at[sidx_v.at[b]], rows[b],
                                  sem_g[b]).wait()
            scale(b)
            pltpu.sync_copy(rows[b], acc_sh.at[didx_v.at[b]], add=True)
            pltpu.async_copy(src_hbm.at[wid, jnp.minimum(ch + 2,
                                                         n_chunks - 1)],
                             sidx_v.at[b], sem_i[b])
            pltpu.async_copy(dst_hbm.at[wid, jnp.minimum(ch + 2,
                                                         n_chunks - 1)],
                             didx_v.at[b], sem_i[b])
            pltpu.async_copy(w_hbm.at[wid, jnp.minimum(ch + 2,
                                                       n_chunks - 1)],
                             wv2.at[b], sem_i[b])

        @pl.loop(0, n_chunks, step=2)
        def _edges(ch):
            body(0, ch)
            body(1, ch + 1)

        pltpu.make_async_copy(hp_hbm.at[sidx_v.at[0]], rows0_v,
                              sem_g0).wait()
        wait_idx(1)
        plsc.subcore_barrier()
        pltpu.sync_copy(
            acc_sh.at[pl.ds(base, rows_per_tile)],
            out_hbm.at[c, pl.ds(base, rows_per_tile)],
        )

    return msg_kernel


def _h_block(x_ref, w_ref, o_ref):
    o_ref[...] = jnp.dot(x_ref[...], w_ref[...],
                         preferred_element_type=jnp.float32)


def _dinv(dall):
    deg = 1.0 + jnp.sum(dall, axis=0)
    return jnp.where(deg > 0, lax.rsqrt(jnp.maximum(deg, 1e-12)), 0.0)


def _hp_block(h_ref, dp_ref, o_ref):
    o_ref[...] = h_ref[...] * _dinv(dp_ref[...])


def _out_block(acc_ref, hp_ref, dp_ref, cb_ref, lwt_ref, lb_ref, o_ref):
    dinv = _dinv(dp_ref[...])
    pre = dinv * (acc_ref[0] + acc_ref[1] + hp_ref[...]) + cb_ref[...]
    r = jnp.maximum(pre, 0.0)
    o_ref[...] = jnp.dot(r, lwt_ref[...],
                         preferred_element_type=jnp.float32) + lb_ref[...]


def kernel(x, edge_index, edge_weight, conv_W, conv_b, lin_W, lin_b):
    n_nodes, d_in = x.shape
    d_hid = conv_W.shape[1]
    d_out = lin_W.shape[0]
    n_edges = edge_weight.shape[0]

    n_chunks = -(-n_edges // (NW * CHUNK))
    n_chunks += n_chunks % 2
    e_pad = NW * n_chunks * CHUNK
    pad = e_pad - n_edges
    n_pad = -(-n_nodes // (NS * 8)) * (NS * 8)

    src = edge_index[0].astype(jnp.int32)
    dst = edge_index[1].astype(jnp.int32)
    w = edge_weight.astype(jnp.float32)
    srcp = jnp.concatenate([src, jnp.zeros((pad,), jnp.int32)]) \
              .reshape(NW, n_chunks, CHUNK)
    dstp = jnp.concatenate([dst, jnp.zeros((pad,), jnp.int32)]) \
              .reshape(NW, n_chunks, CHUNK)
    wp = jnp.concatenate([w, jnp.zeros((pad,), jnp.float32)]) \
            .reshape(NW, n_chunks, CHUNK)

    n_rows = -(-n_nodes // (LANES * CHUNK)) * CHUNK
    degp = _make_deg_kernel(n_rows, n_chunks)(dstp, wp)
    deg2 = degp.reshape(NW, n_rows * LANES)[:, :n_nodes] \
               .reshape(NW, n_nodes, 1)

    blk = 400
    grid = (n_nodes // blk,)
    h = pl.pallas_call(
        _h_block,
        grid=grid,
        in_specs=[
            pl.BlockSpec((blk, d_in), lambda i: (i, 0)),
            pl.BlockSpec((d_in, d_hid), lambda i: (0, 0)),
        ],
        out_specs=pl.BlockSpec((blk, d_hid), lambda i: (i, 0)),
        out_shape=jax.ShapeDtypeStruct((n_nodes, d_hid), jnp.float32),
    )(x, conv_W)

    hp = pl.pallas_call(
        _hp_block,
        grid=grid,
        in_specs=[
            pl.BlockSpec((blk, d_hid), lambda i: (i, 0)),
            pl.BlockSpec((NW, blk, 1), lambda i: (0, i, 0)),
        ],
        out_specs=pl.BlockSpec((blk, d_hid), lambda i: (i, 0)),
        out_shape=jax.ShapeDtypeStruct((n_nodes, d_hid), jnp.float32),
    )(h, deg2)

    acc = _make_msg_kernel(n_pad, d_hid, n_chunks)(hp, srcp, dstp, wp)

    out = pl.pallas_call(
        _out_block,
        grid=grid,
        in_specs=[
            pl.BlockSpec((NC, blk, d_hid), lambda i: (0, i, 0)),
            pl.BlockSpec((blk, d_hid), lambda i: (i, 0)),
            pl.BlockSpec((NW, blk, 1), lambda i: (0, i, 0)),
            pl.BlockSpec((1, d_hid), lambda i: (0, 0)),
            pl.BlockSpec((d_hid, d_out), lambda i: (0, 0)),
            pl.BlockSpec((1, d_out), lambda i: (0, 0)),
        ],
        out_specs=pl.BlockSpec((blk, d_out), lambda i: (i, 0)),
        out_shape=jax.ShapeDtypeStruct((n_nodes, d_out), jnp.float32),
    )(acc, hp, deg2, conv_b.reshape(1, d_hid), lin_W.T,
      lin_b.reshape(1, d_out))

    return out


# speedup vs baseline: 1.0064x; 1.0064x over previous
import functools

import jax
import jax.numpy as jnp
from jax import lax
from jax.experimental import pallas as pl
from jax.experimental.pallas import tpu as pltpu
from jax.experimental.pallas import tpu_sc as plsc

NC = 2
NS = 16
NW = NC * NS
CHUNK = 128
LANES = 16

_mesh = plsc.VectorSubcoreMesh(
    core_axis_name="c", subcore_axis_name="s", num_cores=NC, num_subcores=NS
)
_sc_params = pltpu.CompilerParams(needs_layout_passes=False)


def _make_deg_kernel(n_rows, n_chunks):
    @functools.partial(
        pl.kernel,
        out_type=jax.ShapeDtypeStruct((NW, n_rows, LANES), jnp.float32),
        mesh=_mesh,
        scratch_types=[
            pltpu.VMEM((n_chunks, CHUNK), jnp.int32),
            pltpu.VMEM((n_chunks, CHUNK), jnp.float32),
            pltpu.VMEM((n_rows, LANES), jnp.float32),
        ],
        compiler_params=_sc_params,
    )
    def deg_kernel(dst_hbm, w_hbm, out_hbm, dst_v, w_v, pdeg):
        c = lax.axis_index("c")
        s = lax.axis_index("s")
        wid = s * NC + c
        pltpu.sync_copy(dst_hbm.at[wid], dst_v)
        pltpu.sync_copy(w_hbm.at[wid], w_v)

        zero16 = jnp.zeros((LANES,), jnp.float32)

        @pl.loop(0, n_rows)
        def _zero_pdeg(r):
            pdeg.at[r, pl.ds(0, LANES)][...] = zero16

        @pl.loop(0, n_chunks)
        def _edges(ch):
            for g in range(CHUNK // LANES):
                dst16 = dst_v[ch, pl.ds(g * LANES, LANES)]
                w16 = w_v[ch, pl.ds(g * LANES, LANES)]
                plsc.addupdate_scatter(
                    pdeg, [dst16 >> 4, dst16 & (LANES - 1)], w16)

        pltpu.sync_copy(pdeg, out_hbm.at[wid])

    return deg_kernel


def _make_msg_kernel(n_pad, d, n_chunks):
    rows_per_tile = n_pad // NS

    @functools.partial(
        pl.kernel,
        out_type=jax.ShapeDtypeStruct((NC, n_pad, d), jnp.float32),
        mesh=_mesh,
        scratch_types=[
            pltpu.VMEM((2, CHUNK), jnp.int32),
            pltpu.VMEM((2, CHUNK), jnp.int32),
            pltpu.VMEM((2, CHUNK), jnp.float32),
            pltpu.VMEM((CHUNK, d), jnp.float32),
            pltpu.VMEM((CHUNK, d), jnp.float32),
            pltpu.VMEM_SHARED((n_pad, d), jnp.float32),
            pltpu.SemaphoreType.DMA,
            pltpu.SemaphoreType.DMA,
            pltpu.SemaphoreType.DMA,
            pltpu.SemaphoreType.DMA,
            pltpu.SemaphoreType.DMA,
            pltpu.SemaphoreType.DMA,
            pltpu.SemaphoreType.DMA,
            pltpu.SemaphoreType.DMA,
        ],
        compiler_params=_sc_params,
    )
    def msg_kernel(hp_hbm, src_hbm, dst_hbm, w_hbm, out_hbm,
                   sidx_v, didx_v, wv2, rows0_v, rows1_v, acc_sh,
                   sem_g0, sem_g1, sem_i0, sem_i1, sem_s0, sem_s1,
                   sem_c0, sem_c1):
        c = lax.axis_index("c")
        s = lax.axis_index("s")
        wid = s * NC + c

        zero16 = jnp.zeros((LANES,), jnp.float32)

        @pl.loop(0, CHUNK)
        def _zero_rows(r):
            for q in range(d // LANES):
                rows0_v.at[r, pl.ds(q * LANES, LANES)][...] = zero16

        base = s * rows_per_tile
        off = 0
        while off < rows_per_tile:
            sz = min(CHUNK, rows_per_tile - off)
            pltpu.sync_copy(
                rows0_v.at[pl.ds(0, sz)], acc_sh.at[pl.ds(base + off, sz)]
            )
            off += sz
        plsc.subcore_barrier()

        rows = (rows0_v, rows1_v)
        sem_g = (sem_g0, sem_g1)
        sem_ss = (sem_i0, sem_i1)
        sem_dw = (sem_s0, sem_s1)
        sem_sc = (sem_c0, sem_c1)

        def wait_scatter(b):
            pltpu.make_async_copy(rows[b], acc_sh.at[didx_v.at[b]],
                                  sem_sc[b]).wait()

        def scale(b):
            rw = rows[b]

            @pl.loop(0, CHUNK)
            def _scale(j):
                jidx = jnp.full((LANES,), j, dtype=jnp.int32)
                w16 = plsc.load_gather(wv2.at[b], [jidx])
                for q in range(d // LANES):
                    sl = rw.at[j, pl.ds(q * LANES, LANES)]
                    sl[...] = sl[...] * w16

        pltpu.async_copy(src_hbm.at[wid, 0], sidx_v.at[0], sem_ss[0])
        pltpu.async_copy(src_hbm.at[wid, 1], sidx_v.at[1], sem_ss[1])
        pltpu.async_copy(dst_hbm.at[wid, 0], didx_v.at[0], sem_dw[0])
        pltpu.async_copy(w_hbm.at[wid, 0], wv2.at[0], sem_dw[0])
        pltpu.make_async_copy(src_hbm.at[wid, 0], sidx_v.at[0],
                              sem_ss[0]).wait()
        pltpu.async_copy(hp_hbm.at[sidx_v.at[0]], rows0_v, sem_g0)

        def body(b, ch, first):
            o = 1 - b
            if first:
                pass
            else:
                wait_scatter(o)
            nx1 = jnp.minimum(ch + 1, n_chunks - 1)
            pltpu.async_copy(dst_hbm.at[wid, nx1], didx_v.at[o], sem_dw[o])
            pltpu.async_copy(w_hbm.at[wid, nx1], wv2.at[o], sem_dw[o])
            pltpu.make_async_copy(src_hbm.at[wid, 0], sidx_v.at[o],
                                  sem_ss[o]).wait()
            pltpu.async_copy(hp_hbm.at[sidx_v.at[o]], rows[o], sem_g[o])
            pltpu.make_async_copy(hp_hbm.at[sidx_v.at[b]], rows[b],
                                  sem_g[b]).wait()
            pltpu.async_copy(src_hbm.at[wid, jnp.minimum(ch + 2,
                                                         n_chunks - 1)],
                             sidx_v.at[b], sem_ss[b])
            pltpu.make_async_copy(dst_hbm.at[wid, 0], didx_v.at[b],
                                  sem_dw[b]).wait()
            pltpu.make_async_copy(w_hbm.at[wid, 0], wv2.at[b],
                                  sem_dw[b]).wait()
            scale(b)
            pltpu.async_copy(rows[b], acc_sh.at[didx_v.at[b]], sem_sc[b],
                             add=True)

        body(0, 0, True)
        body(1, 1, False)

        @pl.loop(2, n_chunks, step=2)
        def _edges(ch):
            body(0, ch, False)
            body(1, ch + 1, False)

        wait_scatter(1)
        pltpu.make_async_copy(hp_hbm.at[sidx_v.at[0]], rows0_v,
                              sem_g0).wait()
        pltpu.make_async_copy(src_hbm.at[wid, 0], sidx_v.at[1],
                              sem_ss[1]).wait()
        pltpu.make_async_copy(dst_hbm.at[wid, 0], didx_v.at[0],
                              sem_dw[0]).wait()
        pltpu.make_async_copy(w_hbm.at[wid, 0], wv2.at[0],
                              sem_dw[0]).wait()
        plsc.subcore_barrier()
        pltpu.sync_copy(
            acc_sh.at[pl.ds(base, rows_per_tile)],
            out_hbm.at[c, pl.ds(base, rows_per_tile)],
        )

    return msg_kernel


def _h_block(x_ref, w_ref, o_ref):
    o_ref[...] = jnp.dot(x_ref[...], w_ref[...],
                         preferred_element_type=jnp.float32)


def _dinv(dall):
    deg = 1.0 + jnp.sum(dall, axis=0)
    return jnp.where(deg > 0, lax.rsqrt(jnp.maximum(deg, 1e-12)), 0.0)


def _hp_block(h_ref, dp_ref, o_ref):
    o_ref[...] = h_ref[...] * _dinv(dp_ref[...])


def _out_block(acc_ref, hp_ref, dp_ref, cb_ref, lwt_ref, lb_ref, o_ref):
    dinv = _dinv(dp_ref[...])
    pre = dinv * (acc_ref[0] + acc_ref[1] + hp_ref[...]) + cb_ref[...]
    r = jnp.maximum(pre, 0.0)
    o_ref[...] = jnp.dot(r, lwt_ref[...],
                         preferred_element_type=jnp.float32) + lb_ref[...]


def kernel(x, edge_index, edge_weight, conv_W, conv_b, lin_W, lin_b):
    n_nodes, d_in = x.shape
    d_hid = conv_W.shape[1]
    d_out = lin_W.shape[0]
    n_edges = edge_weight.shape[0]

    n_chunks = -(-n_edges // (NW * CHUNK))
    n_chunks += n_chunks % 2
    e_pad = NW * n_chunks * CHUNK
    pad = e_pad - n_edges
    n_pad = -(-n_nodes // (NS * 8)) * (NS * 8)

    src = edge_index[0].astype(jnp.int32)
    dst = edge_index[1].astype(jnp.int32)
    w = edge_weight.astype(jnp.float32)
    srcp = jnp.concatenate([src, jnp.zeros((pad,), jnp.int32)]) \
              .reshape(NW, n_chunks, CHUNK)
    dstp = jnp.concatenate([dst, jnp.zeros((pad,), jnp.int32)]) \
              .reshape(NW, n_chunks, CHUNK)
    wp = jnp.concatenate([w, jnp.zeros((pad,), jnp.float32)]) \
            .reshape(NW, n_chunks, CHUNK)

    n_rows = -(-n_nodes // (LANES * CHUNK)) * CHUNK
    degp = _make_deg_kernel(n_rows, n_chunks)(dstp, wp)
    deg2 = degp.reshape(NW, n_rows * LANES)[:, :n_nodes] \
               .reshape(NW, n_nodes, 1)

    blk = 400
    grid = (n_nodes // blk,)
    h = pl.pallas_call(
        _h_block,
        grid=grid,
        in_specs=[
            pl.BlockSpec((blk, d_in), lambda i: (i, 0)),
            pl.BlockSpec((d_in, d_hid), lambda i: (0, 0)),
        ],
        out_specs=pl.BlockSpec((blk, d_hid), lambda i: (i, 0)),
        out_shape=jax.ShapeDtypeStruct((n_nodes, d_hid), jnp.float32),
    )(x, conv_W)

    hp = pl.pallas_call(
        _hp_block,
        grid=grid,
        in_specs=[
            pl.BlockSpec((blk, d_hid), lambda i: (i, 0)),
            pl.BlockSpec((NW, blk, 1), lambda i: (0, i, 0)),
        ],
        out_specs=pl.BlockSpec((blk, d_hid), lambda i: (i, 0)),
        out_shape=jax.ShapeDtypeStruct((n_nodes, d_hid), jnp.float32),
    )(h, deg2)

    acc = _make_msg_kernel(n_pad, d_hid, n_chunks)(hp, srcp, dstp, wp)

    out = pl.pallas_call(
        _out_block,
        grid=grid,
        in_specs=[
            pl.BlockSpec((NC, blk, d_hid), lambda i: (0, i, 0)),
            pl.BlockSpec((blk, d_hid), lambda i: (i, 0)),
            pl.BlockSpec((NW, blk, 1), lambda i: (0, i, 0)),
            pl.BlockSpec((1, d_hid), lambda i: (0, 0)),
            pl.BlockSpec((d_hid, d_out), lambda i: (0, 0)),
            pl.BlockSpec((1, d_out), lambda i: (0, 0)),
        ],
        out_specs=pl.BlockSpec((blk, d_out), lambda i: (i, 0)),
        out_shape=jax.ShapeDtypeStruct((n_nodes, d_out), jnp.float32),
    )(acc, hp, deg2, conv_b.reshape(1, d_hid), lin_W.T,
      lin_b.reshape(1, d_out))

    return out


# speedup vs baseline: 1.2511x; 1.2431x over previous
import functools

import jax
import jax.numpy as jnp
from jax import lax
from jax.experimental import pallas as pl
from jax.experimental.pallas import tpu as pltpu
from jax.experimental.pallas import tpu_sc as plsc

NC = 2
NS = 16
NW = NC * NS
CHUNK = 128
LANES = 16

_mesh = plsc.VectorSubcoreMesh(
    core_axis_name="c", subcore_axis_name="s", num_cores=NC, num_subcores=NS
)
_sc_params = pltpu.CompilerParams(needs_layout_passes=False)


def _make_deg_kernel(n_rows, n_chunks):
    @functools.partial(
        pl.kernel,
        out_type=jax.ShapeDtypeStruct((NW, n_rows, LANES), jnp.float32),
        mesh=_mesh,
        scratch_types=[
            pltpu.VMEM((n_chunks, CHUNK), jnp.int32),
            pltpu.VMEM((n_chunks, CHUNK), jnp.float32),
            pltpu.VMEM((n_rows, LANES), jnp.float32),
        ],
        compiler_params=_sc_params,
    )
    def deg_kernel(dst_hbm, w_hbm, out_hbm, dst_v, w_v, pdeg):
        c = lax.axis_index("c")
        s = lax.axis_index("s")
        wid = s * NC + c
        pltpu.sync_copy(dst_hbm.at[wid], dst_v)
        pltpu.sync_copy(w_hbm.at[wid], w_v)

        zero16 = jnp.zeros((LANES,), jnp.float32)

        @pl.loop(0, n_rows)
        def _zero_pdeg(r):
            pdeg.at[r, pl.ds(0, LANES)][...] = zero16

        @pl.loop(0, n_chunks)
        def _edges(ch):
            for g in range(CHUNK // LANES):
                dst16 = dst_v[ch, pl.ds(g * LANES, LANES)]
                w16 = w_v[ch, pl.ds(g * LANES, LANES)]
                plsc.addupdate_scatter(
                    pdeg, [dst16 >> 4, dst16 & (LANES - 1)], w16)

        pltpu.sync_copy(pdeg, out_hbm.at[wid])

    return deg_kernel


def _make_msg_kernel(n_pad, d, n_chunks):
    rows_per_tile = n_pad // NS

    @functools.partial(
        pl.kernel,
        out_type=jax.ShapeDtypeStruct((NC, n_pad, d), jnp.float32),
        mesh=_mesh,
        scratch_types=[
            pltpu.VMEM((n_chunks, CHUNK), jnp.int32),
            pltpu.VMEM((n_chunks, CHUNK), jnp.int32),
            pltpu.VMEM((n_chunks, CHUNK), jnp.float32),
            pltpu.VMEM((CHUNK, d), jnp.float32),
            pltpu.VMEM_SHARED((n_pad, d), jnp.float32),
            pltpu.SemaphoreType.DMA,
        ],
        compiler_params=_sc_params,
    )
    def msg_kernel(hp_hbm, src_hbm, dst_hbm, w_hbm, out_hbm,
                   src_v, dst_v, w_v, rows0_v, acc_sh, sem0):
        c = lax.axis_index("c")
        s = lax.axis_index("s")
        wid = s * NC + c
        pltpu.sync_copy(src_hbm.at[wid], src_v)
        pltpu.sync_copy(dst_hbm.at[wid], dst_v)
        pltpu.sync_copy(w_hbm.at[wid], w_v)

        zero16 = jnp.zeros((LANES,), jnp.float32)

        @pl.loop(0, CHUNK)
        def _zero_rows(r):
            for q in range(d // LANES):
                rows0_v.at[r, pl.ds(q * LANES, LANES)][...] = zero16

        base = s * rows_per_tile
        off = 0
        while off < rows_per_tile:
            sz = min(CHUNK, rows_per_tile - off)
            pltpu.sync_copy(
                rows0_v.at[pl.ds(0, sz)], acc_sh.at[pl.ds(base + off, sz)]
            )
            off += sz
        plsc.subcore_barrier()

        def scale(rows, ch):
            @pl.loop(0, CHUNK)
            def _scale(j):
                jidx = jnp.full((LANES,), j, dtype=jnp.int32)
                w16 = plsc.load_gather(w_v.at[ch], [jidx])
                for q in range(d // LANES):
                    sl = rows.at[j, pl.ds(q * LANES, LANES)]
                    sl[...] = sl[...] * w16

        @pl.loop(0, n_chunks)
        def _edges(ch):
            pltpu.async_copy(hp_hbm.at[src_v.at[ch]], rows0_v, sem0).wait()
            scale(rows0_v, ch)
            pltpu.sync_copy(rows0_v, acc_sh.at[dst_v.at[ch]], add=True)

        plsc.subcore_barrier()
        pltpu.sync_copy(
            acc_sh.at[pl.ds(base, rows_per_tile)],
            out_hbm.at[c, pl.ds(base, rows_per_tile)],
        )

    return msg_kernel


def _h_block(x_ref, w_ref, o_ref):
    o_ref[...] = jnp.dot(x_ref[...], w_ref[...],
                         preferred_element_type=jnp.float32)


def _dinv(dall):
    deg = 1.0 + jnp.sum(dall, axis=0)
    return jnp.where(deg > 0, lax.rsqrt(jnp.maximum(deg, 1e-12)), 0.0)


def _hp_block(h_ref, dp_ref, o_ref):
    o_ref[...] = h_ref[...] * _dinv(dp_ref[...])


def _out_block(acc_ref, hp_ref, dp_ref, cb_ref, lwt_ref, lb_ref, o_ref):
    dinv = _dinv(dp_ref[...])
    pre = dinv * (acc_ref[0] + acc_ref[1] + hp_ref[...]) + cb_ref[...]
    r = jnp.maximum(pre, 0.0)
    o_ref[...] = jnp.dot(r, lwt_ref[...],
                         preferred_element_type=jnp.float32) + lb_ref[...]


def kernel(x, edge_index, edge_weight, conv_W, conv_b, lin_W, lin_b):
    n_nodes, d_in = x.shape
    d_hid = conv_W.shape[1]
    d_out = lin_W.shape[0]
    n_edges = edge_weight.shape[0]

    n_chunks = -(-n_edges // (NW * CHUNK))
    e_pad = NW * n_chunks * CHUNK
    pad = e_pad - n_edges
    n_pad = -(-n_nodes // (NS * 8)) * (NS * 8)

    src = edge_index[0].astype(jnp.int32)
    dst = edge_index[1].astype(jnp.int32)
    w = edge_weight.astype(jnp.float32)
    srcp = jnp.concatenate([src, jnp.zeros((pad,), jnp.int32)]) \
              .reshape(NW, n_chunks, CHUNK)
    dstp = jnp.concatenate([dst, jnp.zeros((pad,), jnp.int32)]) \
              .reshape(NW, n_chunks, CHUNK)
    wp = jnp.concatenate([w, jnp.zeros((pad,), jnp.float32)]) \
            .reshape(NW, n_chunks, CHUNK)

    n_rows = -(-n_nodes // (LANES * CHUNK)) * CHUNK
    degp = _make_deg_kernel(n_rows, n_chunks)(dstp, wp)
    deg2 = degp.reshape(NW, n_rows * LANES)[:, :n_nodes] \
               .reshape(NW, n_nodes, 1)

    blk = 400
    grid = (n_nodes // blk,)
    h = pl.pallas_call(
        _h_block,
        grid=grid,
        in_specs=[
            pl.BlockSpec((blk, d_in), lambda i: (i, 0)),
            pl.BlockSpec((d_in, d_hid), lambda i: (0, 0)),
        ],
        out_specs=pl.BlockSpec((blk, d_hid), lambda i: (i, 0)),
        out_shape=jax.ShapeDtypeStruct((n_nodes, d_hid), jnp.float32),
    )(x, conv_W)

    hp = pl.pallas_call(
        _hp_block,
        grid=grid,
        in_specs=[
            pl.BlockSpec((blk, d_hid), lambda i: (i, 0)),
            pl.BlockSpec((NW, blk, 1), lambda i: (0, i, 0)),
        ],
        out_specs=pl.BlockSpec((blk, d_hid), lambda i: (i, 0)),
        out_shape=jax.ShapeDtypeStruct((n_nodes, d_hid), jnp.float32),
    )(h, deg2)

    acc = _make_msg_kernel(n_pad, d_hid, n_chunks)(hp, srcp, dstp, wp)

    out = pl.pallas_call(
        _out_block,
        grid=grid,
        in_specs=[
            pl.BlockSpec((NC, blk, d_hid), lambda i: (0, i, 0)),
            pl.BlockSpec((blk, d_hid), lambda i: (i, 0)),
            pl.BlockSpec((NW, blk, 1), lambda i: (0, i, 0)),
            pl.BlockSpec((1, d_hid), lambda i: (0, 0)),
            pl.BlockSpec((d_hid, d_out), lambda i: (0, 0)),
            pl.BlockSpec((1, d_out), lambda i: (0, 0)),
        ],
        out_specs=pl.BlockSpec((blk, d_out), lambda i: (i, 0)),
        out_shape=jax.ShapeDtypeStruct((n_nodes, d_out), jnp.float32),
    )(acc, hp, deg2, conv_b.reshape(1, d_hid), lin_W.T,
      lin_b.reshape(1, d_out))

    return out
